# baseline (speedup 1.0000x reference)
import jax
import jax.numpy as jnp
from jax import lax
from jax.experimental import pallas as pl
from jax.experimental.pallas import tpu as pltpu, tpu_sc as plsc
import functools

NUM_CORES = 2
NUM_SUBCORES = 16
NW = NUM_CORES * NUM_SUBCORES
BATCH = 16384
D = 128
Q = D // 4
B_PER_W = BATCH // NW
CH = 128
NCH = B_PER_W // CH

_mesh = plsc.VectorSubcoreMesh(core_axis_name="c", subcore_axis_name="s")


@functools.partial(
    pl.kernel,
    mesh=_mesh,
    out_type=jax.ShapeDtypeStruct((BATCH,), jnp.float32),
    scratch_types=[
        pltpu.VMEM((NCH, CH), jnp.int32),
        pltpu.VMEM((NCH, CH), jnp.int32),
        pltpu.VMEM((NCH, CH), jnp.int32),
        pltpu.VMEM((2, CH, D), jnp.float32),
        pltpu.VMEM((2, CH, D), jnp.float32),
        pltpu.VMEM((2, CH, D), jnp.float32),
        pltpu.VMEM((CH,), jnp.float32),
        pltpu.SemaphoreType.DMA,
        pltpu.SemaphoreType.DMA,
    ],
)
def _quate_sc(head_hbm, rel_hbm, tail_hbm, ent_hbm, relemb_hbm, out_hbm,
              idx_h, idx_r, idx_t, h_buf, r_buf, t_buf,
              score_buf, sem0, sem1):
    wid = lax.axis_index("s") * NUM_CORES + lax.axis_index("c")
    pltpu.sync_copy(head_hbm.at[wid], idx_h)
    pltpu.sync_copy(rel_hbm.at[wid], idx_r)
    pltpu.sync_copy(tail_hbm.at[wid], idx_t)

    def issue(j, slot, sem):
        pltpu.async_copy(ent_hbm.at[idx_h.at[j]], h_buf.at[slot], sem)
        pltpu.async_copy(relemb_hbm.at[idx_r.at[j]], r_buf.at[slot], sem)
        pltpu.async_copy(ent_hbm.at[idx_t.at[j]], t_buf.at[slot], sem)

    def drain(j, slot, sem):
        pltpu.make_async_copy(ent_hbm.at[idx_h.at[j]], h_buf.at[slot], sem).wait()
        pltpu.make_async_copy(relemb_hbm.at[idx_r.at[j]], r_buf.at[slot], sem).wait()
        pltpu.make_async_copy(ent_hbm.at[idx_t.at[j]], t_buf.at[slot], sem).wait()

    issue(0, 0, sem0)

    def do_chunk(j, _):
        par = j % 2

        @pl.when(par == 0)
        def _():
            drain(j, 0, sem0)

        @pl.when(par == 1)
        def _():
            drain(j, 1, sem1)

        @pl.when((par == 0) & (j + 1 < NCH))
        def _():
            issue(j + 1, 1, sem1)

        @pl.when((par == 1) & (j + 1 < NCH))
        def _():
            issue(j + 1, 0, sem0)

        @plsc.parallel_loop(0, CH // 16, unroll=2)
        def do_group(g):
            base = g * 16
            lane = lax.iota(jnp.int32, 16)
            gvec = jnp.zeros((16,), jnp.float32)
            for rr in range(16):
                row = base + rr
                acc = jnp.zeros((16,), jnp.float32)
                for v in range(2):
                    hi = h_buf[par, row, pl.ds(0 * Q + v * 16, 16)]
                    hj = h_buf[par, row, pl.ds(1 * Q + v * 16, 16)]
                    hk = h_buf[par, row, pl.ds(2 * Q + v * 16, 16)]
                    hl = h_buf[par, row, pl.ds(3 * Q + v * 16, 16)]
                    ri = r_buf[par, row, pl.ds(0 * Q + v * 16, 16)]
                    rj = r_buf[par, row, pl.ds(1 * Q + v * 16, 16)]
                    rk = r_buf[par, row, pl.ds(2 * Q + v * 16, 16)]
                    rl = r_buf[par, row, pl.ds(3 * Q + v * 16, 16)]
                    ti = t_buf[par, row, pl.ds(0 * Q + v * 16, 16)]
                    tj = t_buf[par, row, pl.ds(1 * Q + v * 16, 16)]
                    tk = t_buf[par, row, pl.ds(2 * Q + v * 16, 16)]
                    tl = t_buf[par, row, pl.ds(3 * Q + v * 16, 16)]
                    ca = ri * ti + rj * tj + rk * tk + rl * tl
                    cb = ri * tj - rj * ti + rk * tl - rl * tk
                    cc = ri * tk - rj * tl - rk * ti + rl * tj
                    cd = ri * tl + rj * tk - rk * tj - rl * ti
                    acc = acc + hi * ca + hj * cb + hk * cc + hl * cd
                s = acc[0]
                for c in range(1, 16):
                    s = s + acc[c]
                gvec = jnp.where(lane == rr, s, gvec)
            score_buf[pl.ds(base, 16)] = gvec

        pltpu.sync_copy(score_buf, out_hbm.at[pl.ds(wid * B_PER_W + j * CH, CH)])
        return 0

    lax.fori_loop(0, NCH, do_chunk, 0)


def kernel(head, relation, tail, entity_emb, relation_emb):
    head3 = head.astype(jnp.int32).reshape(NW, NCH, CH)
    rel3 = relation.astype(jnp.int32).reshape(NW, NCH, CH)
    tail3 = tail.astype(jnp.int32).reshape(NW, NCH, CH)
    return _quate_sc(head3, rel3, tail3, entity_emb, relation_emb)


# speedup vs baseline: 1.6955x; 1.6955x over previous
import jax
import jax.numpy as jnp
from jax import lax
from jax.experimental import pallas as pl
from jax.experimental.pallas import tpu as pltpu, tpu_sc as plsc
import functools

NUM_CORES = 2
NUM_SUBCORES = 16
NW = NUM_CORES * NUM_SUBCORES
BATCH = 16384
D = 128
Q = D // 4
B_PER_W = BATCH // NW
CH = 128
NCH = B_PER_W // CH

_mesh = plsc.VectorSubcoreMesh(core_axis_name="c", subcore_axis_name="s")


@functools.partial(
    pl.kernel,
    mesh=_mesh,
    out_type=jax.ShapeDtypeStruct((BATCH,), jnp.float32),
    scratch_types=[
        pltpu.VMEM((NCH, CH), jnp.int32),
        pltpu.VMEM((NCH, CH), jnp.int32),
        pltpu.VMEM((NCH, CH), jnp.int32),
        pltpu.VMEM((2, CH, D), jnp.float32),
        pltpu.VMEM((2, CH, D), jnp.float32),
        pltpu.VMEM((2, CH, D), jnp.float32),
        pltpu.VMEM((CH,), jnp.float32),
        pltpu.SemaphoreType.DMA,
        pltpu.SemaphoreType.DMA,
    ],
)
def _quate_sc(head_hbm, rel_hbm, tail_hbm, ent_hbm, relemb_hbm, out_hbm,
              idx_h, idx_r, idx_t, h_buf, r_buf, t_buf,
              score_buf, sem0, sem1):
    wid = lax.axis_index("s") * NUM_CORES + lax.axis_index("c")
    pltpu.sync_copy(head_hbm.at[wid], idx_h)
    pltpu.sync_copy(rel_hbm.at[wid], idx_r)
    pltpu.sync_copy(tail_hbm.at[wid], idx_t)

    def issue(j, slot, sem):
        pltpu.async_copy(ent_hbm.at[idx_h.at[j]], h_buf.at[slot], sem)
        pltpu.async_copy(relemb_hbm.at[idx_r.at[j]], r_buf.at[slot], sem)
        pltpu.async_copy(ent_hbm.at[idx_t.at[j]], t_buf.at[slot], sem)

    def drain(j, slot, sem):
        pltpu.make_async_copy(ent_hbm.at[idx_h.at[j]], h_buf.at[slot], sem).wait()
        pltpu.make_async_copy(relemb_hbm.at[idx_r.at[j]], r_buf.at[slot], sem).wait()
        pltpu.make_async_copy(ent_hbm.at[idx_t.at[j]], t_buf.at[slot], sem).wait()

    issue(0, 0, sem0)

    def do_chunk(j, _):
        par = j % 2

        @pl.when(par == 0)
        def _():
            drain(j, 0, sem0)

        @pl.when(par == 1)
        def _():
            drain(j, 1, sem1)

        @pl.when((par == 0) & (j + 1 < NCH))
        def _():
            issue(j + 1, 1, sem1)

        @pl.when((par == 1) & (j + 1 < NCH))
        def _():
            issue(j + 1, 0, sem0)

        @plsc.parallel_loop(0, CH // 16)
        def do_group(g):
            base = g * 16
            lane = lax.iota(jnp.int32, 16)
            gvec = jnp.zeros((16,), jnp.float32)
            for rr in range(16):
                row = base + rr
                acc = jnp.zeros((16,), jnp.float32)
                for v in range(2):
                    hi = h_buf[par, row, pl.ds(0 * Q + v * 16, 16)]
                    hj = h_buf[par, row, pl.ds(1 * Q + v * 16, 16)]
                    hk = h_buf[par, row, pl.ds(2 * Q + v * 16, 16)]
                    hl = h_buf[par, row, pl.ds(3 * Q + v * 16, 16)]
                    ri = r_buf[par, row, pl.ds(0 * Q + v * 16, 16)]
                    rj = r_buf[par, row, pl.ds(1 * Q + v * 16, 16)]
                    rk = r_buf[par, row, pl.ds(2 * Q + v * 16, 16)]
                    rl = r_buf[par, row, pl.ds(3 * Q + v * 16, 16)]
                    ti = t_buf[par, row, pl.ds(0 * Q + v * 16, 16)]
                    tj = t_buf[par, row, pl.ds(1 * Q + v * 16, 16)]
                    tk = t_buf[par, row, pl.ds(2 * Q + v * 16, 16)]
                    tl = t_buf[par, row, pl.ds(3 * Q + v * 16, 16)]
                    ca = ri * ti + rj * tj + rk * tk + rl * tl
                    cb = ri * tj - rj * ti + rk * tl - rl * tk
                    cc = ri * tk - rj * tl - rk * ti + rl * tj
                    cd = ri * tl + rj * tk - rk * tj - rl * ti
                    acc = acc + hi * ca + hj * cb + hk * cc + hl * cd
                s = acc[0]
                for c in range(1, 16):
                    s = s + acc[c]
                gvec = jnp.where(lane == rr, s, gvec)
            score_buf[pl.ds(base, 16)] = gvec

        pltpu.sync_copy(score_buf, out_hbm.at[pl.ds(wid * B_PER_W + j * CH, CH)])
        return 0

    lax.fori_loop(0, NCH, do_chunk, 0)


def kernel(head, relation, tail, entity_emb, relation_emb):
    head3 = head.astype(jnp.int32).reshape(NW, NCH, CH)
    rel3 = relation.astype(jnp.int32).reshape(NW, NCH, CH)
    tail3 = tail.astype(jnp.int32).reshape(NW, NCH, CH)
    return _quate_sc(head3, rel3, tail3, entity_emb, relation_emb)


# speedup vs baseline: 1.6961x; 1.0004x over previous
import jax
import jax.numpy as jnp
from jax import lax
from jax.experimental import pallas as pl
from jax.experimental.pallas import tpu as pltpu, tpu_sc as plsc
import functools

NUM_CORES = 2
NUM_SUBCORES = 16
NW = NUM_CORES * NUM_SUBCORES
BATCH = 16384
D = 128
Q = D // 4
B_PER_W = BATCH // NW
CH = 128
NCH = B_PER_W // CH

_mesh = plsc.VectorSubcoreMesh(core_axis_name="c", subcore_axis_name="s")


@functools.partial(
    pl.kernel,
    mesh=_mesh,
    out_type=jax.ShapeDtypeStruct((BATCH,), jnp.float32),
    scratch_types=[
        pltpu.VMEM((NCH, CH), jnp.int32),
        pltpu.VMEM((NCH, CH), jnp.int32),
        pltpu.VMEM((NCH, CH), jnp.int32),
        pltpu.VMEM((2, CH, D), jnp.float32),
        pltpu.VMEM((2, CH, D), jnp.float32),
        pltpu.VMEM((2, CH, D), jnp.float32),
        pltpu.VMEM((16, 32), jnp.float32),
        pltpu.VMEM((CH,), jnp.float32),
        pltpu.SemaphoreType.DMA,
        pltpu.SemaphoreType.DMA,
    ],
)
def _quate_sc(head_hbm, rel_hbm, tail_hbm, ent_hbm, relemb_hbm, out_hbm,
              idx_h, idx_r, idx_t, h_buf, r_buf, t_buf, pad,
              score_buf, sem0, sem1):
    wid = lax.axis_index("s") * NUM_CORES + lax.axis_index("c")
    pltpu.sync_copy(head_hbm.at[wid], idx_h)
    pltpu.sync_copy(rel_hbm.at[wid], idx_r)
    pltpu.sync_copy(tail_hbm.at[wid], idx_t)

    def issue(j, slot, sem):
        pltpu.async_copy(ent_hbm.at[idx_h.at[j]], h_buf.at[slot], sem)
        pltpu.async_copy(relemb_hbm.at[idx_r.at[j]], r_buf.at[slot], sem)
        pltpu.async_copy(ent_hbm.at[idx_t.at[j]], t_buf.at[slot], sem)

    def drain(j, slot, sem):
        pltpu.make_async_copy(ent_hbm.at[idx_h.at[j]], h_buf.at[slot], sem).wait()
        pltpu.make_async_copy(relemb_hbm.at[idx_r.at[j]], r_buf.at[slot], sem).wait()
        pltpu.make_async_copy(ent_hbm.at[idx_t.at[j]], t_buf.at[slot], sem).wait()

    issue(0, 0, sem0)

    def do_chunk(j, _):
        par = j % 2

        @pl.when(par == 0)
        def _():
            drain(j, 0, sem0)

        @pl.when(par == 1)
        def _():
            drain(j, 1, sem1)

        @pl.when((par == 0) & (j + 1 < NCH))
        def _():
            issue(j + 1, 1, sem1)

        @pl.when((par == 1) & (j + 1 < NCH))
        def _():
            issue(j + 1, 0, sem0)

        @plsc.parallel_loop(0, CH // 16)
        def do_group(g):
            base = g * 16
            lane = lax.iota(jnp.int32, 16)
            gvec = jnp.zeros((16,), jnp.float32)
            for rr in range(16):
                row = base + rr
                acc = jnp.zeros((16,), jnp.float32)
                for v in range(2):
                    hi = h_buf[par, row, pl.ds(0 * Q + v * 16, 16)]
                    hj = h_buf[par, row, pl.ds(1 * Q + v * 16, 16)]
                    hk = h_buf[par, row, pl.ds(2 * Q + v * 16, 16)]
                    hl = h_buf[par, row, pl.ds(3 * Q + v * 16, 16)]
                    ri = r_buf[par, row, pl.ds(0 * Q + v * 16, 16)]
                    rj = r_buf[par, row, pl.ds(1 * Q + v * 16, 16)]
                    rk = r_buf[par, row, pl.ds(2 * Q + v * 16, 16)]
                    rl = r_buf[par, row, pl.ds(3 * Q + v * 16, 16)]
                    ti = t_buf[par, row, pl.ds(0 * Q + v * 16, 16)]
                    tj = t_buf[par, row, pl.ds(1 * Q + v * 16, 16)]
                    tk = t_buf[par, row, pl.ds(2 * Q + v * 16, 16)]
                    tl = t_buf[par, row, pl.ds(3 * Q + v * 16, 16)]
                    ca = ri * ti + rj * tj + rk * tk + rl * tl
                    cb = ri * tj - rj * ti + rk * tl - rl * tk
                    cc = ri * tk - rj * tl - rk * ti + rl * tj
                    cd = ri * tl + rj * tk - rk * tj - rl * ti
                    acc = acc + hi * ca + hj * cb + hk * cc + hl * cd
                pad[rr, pl.ds(0, 16)] = acc
                acc2 = acc + pad[rr, pl.ds(8, 16)]
                s = acc2[0]
                for c in range(1, 8):
                    s = s + acc2[c]
                gvec = jnp.where(lane == rr, s, gvec)
            score_buf[pl.ds(base, 16)] = gvec

        pltpu.sync_copy(score_buf, out_hbm.at[pl.ds(wid * B_PER_W + j * CH, CH)])
        return 0

    lax.fori_loop(0, NCH, do_chunk, 0)


def kernel(head, relation, tail, entity_emb, relation_emb):
    head3 = head.astype(jnp.int32).reshape(NW, NCH, CH)
    rel3 = relation.astype(jnp.int32).reshape(NW, NCH, CH)
    tail3 = tail.astype(jnp.int32).reshape(NW, NCH, CH)
    return _quate_sc(head3, rel3, tail3, entity_emb, relation_emb)


# speedup vs baseline: 1.7371x; 1.0242x over previous
import jax
import jax.numpy as jnp
from jax import lax
from jax.experimental import pallas as pl
from jax.experimental.pallas import tpu as pltpu, tpu_sc as plsc
import functools

NUM_CORES = 2
NUM_SUBCORES = 16
NW = NUM_CORES * NUM_SUBCORES
BATCH = 16384
D = 128
Q = D // 4
B_PER_W = BATCH // NW
CH = 128
NCH = B_PER_W // CH

_mesh = plsc.VectorSubcoreMesh(core_axis_name="c", subcore_axis_name="s")


@functools.partial(
    pl.kernel,
    mesh=_mesh,
    out_type=jax.ShapeDtypeStruct((BATCH,), jnp.float32),
    scratch_types=[
        pltpu.VMEM((B_PER_W,), jnp.int32),
        pltpu.VMEM((B_PER_W,), jnp.int32),
        pltpu.VMEM((B_PER_W,), jnp.int32),
        pltpu.VMEM((2, CH, D), jnp.float32),
        pltpu.VMEM((2, CH, D), jnp.float32),
        pltpu.VMEM((2, CH, D), jnp.float32),
        pltpu.VMEM((16, 32), jnp.float32),
        pltpu.VMEM((CH,), jnp.float32),
        pltpu.SemaphoreType.DMA,
        pltpu.SemaphoreType.DMA,
    ],
)
def _quate_sc(head_hbm, rel_hbm, tail_hbm, ent_hbm, relemb_hbm, out_hbm,
              idx_h, idx_r, idx_t, h_buf, r_buf, t_buf, pad,
              score_buf, sem0, sem1):
    wid = lax.axis_index("s") * NUM_CORES + lax.axis_index("c")
    ibase = wid * B_PER_W
    pltpu.async_copy(head_hbm.at[pl.ds(ibase, B_PER_W)], idx_h, sem0)
    pltpu.async_copy(rel_hbm.at[pl.ds(ibase, B_PER_W)], idx_r, sem0)
    pltpu.async_copy(tail_hbm.at[pl.ds(ibase, B_PER_W)], idx_t, sem0)
    pltpu.make_async_copy(head_hbm.at[pl.ds(ibase, B_PER_W)], idx_h, sem0).wait()
    pltpu.make_async_copy(rel_hbm.at[pl.ds(ibase, B_PER_W)], idx_r, sem0).wait()
    pltpu.make_async_copy(tail_hbm.at[pl.ds(ibase, B_PER_W)], idx_t, sem0).wait()

    def issue(j, slot, sem):
        sl = pl.ds(j * CH, CH)
        pltpu.async_copy(ent_hbm.at[idx_h.at[sl]], h_buf.at[slot], sem)
        pltpu.async_copy(relemb_hbm.at[idx_r.at[sl]], r_buf.at[slot], sem)
        pltpu.async_copy(ent_hbm.at[idx_t.at[sl]], t_buf.at[slot], sem)

    def drain(j, slot, sem):
        sl = pl.ds(j * CH, CH)
        pltpu.make_async_copy(ent_hbm.at[idx_h.at[sl]], h_buf.at[slot], sem).wait()
        pltpu.make_async_copy(relemb_hbm.at[idx_r.at[sl]], r_buf.at[slot], sem).wait()
        pltpu.make_async_copy(ent_hbm.at[idx_t.at[sl]], t_buf.at[slot], sem).wait()

    issue(0, 0, sem0)

    def do_chunk(j, _):
        par = j % 2

        @pl.when(par == 0)
        def _():
            drain(j, 0, sem0)

        @pl.when(par == 1)
        def _():
            drain(j, 1, sem1)

        @pl.when((par == 0) & (j + 1 < NCH))
        def _():
            issue(j + 1, 1, sem1)

        @pl.when((par == 1) & (j + 1 < NCH))
        def _():
            issue(j + 1, 0, sem0)

        @plsc.parallel_loop(0, CH // 16)
        def do_group(g):
            base = g * 16
            lane = lax.iota(jnp.int32, 16)
            gvec = jnp.zeros((16,), jnp.float32)
            for rr in range(16):
                row = base + rr
                acc = jnp.zeros((16,), jnp.float32)
                for v in range(2):
                    hi = h_buf[par, row, pl.ds(0 * Q + v * 16, 16)]
                    hj = h_buf[par, row, pl.ds(1 * Q + v * 16, 16)]
                    hk = h_buf[par, row, pl.ds(2 * Q + v * 16, 16)]
                    hl = h_buf[par, row, pl.ds(3 * Q + v * 16, 16)]
                    ri = r_buf[par, row, pl.ds(0 * Q + v * 16, 16)]
                    rj = r_buf[par, row, pl.ds(1 * Q + v * 16, 16)]
                    rk = r_buf[par, row, pl.ds(2 * Q + v * 16, 16)]
                    rl = r_buf[par, row, pl.ds(3 * Q + v * 16, 16)]
                    ti = t_buf[par, row, pl.ds(0 * Q + v * 16, 16)]
                    tj = t_buf[par, row, pl.ds(1 * Q + v * 16, 16)]
                    tk = t_buf[par, row, pl.ds(2 * Q + v * 16, 16)]
                    tl = t_buf[par, row, pl.ds(3 * Q + v * 16, 16)]
                    ca = ri * ti + rj * tj + rk * tk + rl * tl
                    cb = ri * tj - rj * ti + rk * tl - rl * tk
                    cc = ri * tk - rj * tl - rk * ti + rl * tj
                    cd = ri * tl + rj * tk - rk * tj - rl * ti
                    acc = acc + hi * ca + hj * cb + hk * cc + hl * cd
                pad[rr, pl.ds(0, 16)] = acc
                acc2 = acc + pad[rr, pl.ds(8, 16)]
                s = acc2[0]
                for c in range(1, 8):
                    s = s + acc2[c]
                gvec = jnp.where(lane == rr, s, gvec)
            score_buf[pl.ds(base, 16)] = gvec

        pltpu.sync_copy(score_buf, out_hbm.at[pl.ds(wid * B_PER_W + j * CH, CH)])
        return 0

    lax.fori_loop(0, NCH, do_chunk, 0)


def kernel(head, relation, tail, entity_emb, relation_emb):
    return _quate_sc(head.astype(jnp.int32), relation.astype(jnp.int32),
                     tail.astype(jnp.int32), entity_emb, relation_emb)


# speedup vs baseline: 1.9496x; 1.1223x over previous
import jax
import jax.numpy as jnp
from jax import lax
from jax.experimental import pallas as pl
from jax.experimental.pallas import tpu as pltpu, tpu_sc as plsc
import functools

NUM_CORES = 2
NUM_SUBCORES = 16
NW = NUM_CORES * NUM_SUBCORES
BATCH = 16384
D = 128
Q = D // 4
B_PER_W = BATCH // NW
CH = 128
NCH = B_PER_W // CH

_mesh = plsc.VectorSubcoreMesh(core_axis_name="c", subcore_axis_name="s")


@functools.partial(
    pl.kernel,
    mesh=_mesh,
    out_type=jax.ShapeDtypeStruct((BATCH,), jnp.float32),
    scratch_types=[
        pltpu.VMEM((B_PER_W,), jnp.int32),
        pltpu.VMEM((B_PER_W,), jnp.int32),
        pltpu.VMEM((B_PER_W,), jnp.int32),
        pltpu.VMEM((2, CH, D), jnp.float32),
        pltpu.VMEM((2, CH, D), jnp.float32),
        pltpu.VMEM((2, CH, D), jnp.float32),
        pltpu.VMEM((CH, 32), jnp.float32),
        pltpu.VMEM((CH,), jnp.float32),
        pltpu.SemaphoreType.DMA,
        pltpu.SemaphoreType.DMA,
    ],
)
def _quate_sc(head_hbm, rel_hbm, tail_hbm, ent_hbm, relemb_hbm, out_hbm,
              idx_h, idx_r, idx_t, h_buf, r_buf, t_buf, pad,
              score_buf, sem0, sem1):
    wid = lax.axis_index("s") * NUM_CORES + lax.axis_index("c")
    ibase = wid * B_PER_W
    pltpu.async_copy(head_hbm.at[pl.ds(ibase, B_PER_W)], idx_h, sem0)
    pltpu.async_copy(rel_hbm.at[pl.ds(ibase, B_PER_W)], idx_r, sem0)
    pltpu.async_copy(tail_hbm.at[pl.ds(ibase, B_PER_W)], idx_t, sem0)
    pltpu.make_async_copy(head_hbm.at[pl.ds(ibase, B_PER_W)], idx_h, sem0).wait()
    pltpu.make_async_copy(rel_hbm.at[pl.ds(ibase, B_PER_W)], idx_r, sem0).wait()
    pltpu.make_async_copy(tail_hbm.at[pl.ds(ibase, B_PER_W)], idx_t, sem0).wait()

    def issue(j, slot, sem):
        sl = pl.ds(j * CH, CH)
        pltpu.async_copy(ent_hbm.at[idx_h.at[sl]], h_buf.at[slot], sem)
        pltpu.async_copy(relemb_hbm.at[idx_r.at[sl]], r_buf.at[slot], sem)
        pltpu.async_copy(ent_hbm.at[idx_t.at[sl]], t_buf.at[slot], sem)

    def drain(j, slot, sem):
        sl = pl.ds(j * CH, CH)
        pltpu.make_async_copy(ent_hbm.at[idx_h.at[sl]], h_buf.at[slot], sem).wait()
        pltpu.make_async_copy(relemb_hbm.at[idx_r.at[sl]], r_buf.at[slot], sem).wait()
        pltpu.make_async_copy(ent_hbm.at[idx_t.at[sl]], t_buf.at[slot], sem).wait()

    issue(0, 0, sem0)

    def do_chunk(j, _):
        par = j % 2

        @pl.when(par == 0)
        def _():
            drain(j, 0, sem0)

        @pl.when(par == 1)
        def _():
            drain(j, 1, sem1)

        @pl.when((par == 0) & (j + 1 < NCH))
        def _():
            issue(j + 1, 1, sem1)

        @pl.when((par == 1) & (j + 1 < NCH))
        def _():
            issue(j + 1, 0, sem0)

        lane = lax.iota(jnp.int32, 16)
        zeros16 = jnp.zeros((16,), jnp.float32)
        for m in range(CH // 16):
            score_buf[pl.ds(m * 16, 16)] = zeros16

        @plsc.parallel_loop(0, CH, 1, unroll=2)
        def do_row(row):
            acc = jnp.zeros((16,), jnp.float32)
            for v in range(2):
                hi = h_buf[par, row, pl.ds(0 * Q + v * 16, 16)]
                hj = h_buf[par, row, pl.ds(1 * Q + v * 16, 16)]
                hk = h_buf[par, row, pl.ds(2 * Q + v * 16, 16)]
                hl = h_buf[par, row, pl.ds(3 * Q + v * 16, 16)]
                ri = r_buf[par, row, pl.ds(0 * Q + v * 16, 16)]
                rj = r_buf[par, row, pl.ds(1 * Q + v * 16, 16)]
                rk = r_buf[par, row, pl.ds(2 * Q + v * 16, 16)]
                rl = r_buf[par, row, pl.ds(3 * Q + v * 16, 16)]
                ti = t_buf[par, row, pl.ds(0 * Q + v * 16, 16)]
                tj = t_buf[par, row, pl.ds(1 * Q + v * 16, 16)]
                tk = t_buf[par, row, pl.ds(2 * Q + v * 16, 16)]
                tl = t_buf[par, row, pl.ds(3 * Q + v * 16, 16)]
                ca = ri * ti + rj * tj + rk * tk + rl * tl
                cb = ri * tj - rj * ti + rk * tl - rl * tk
                cc = ri * tk - rj * tl - rk * ti + rl * tj
                cd = ri * tl + rj * tk - rk * tj - rl * ti
                acc = acc + hi * ca + hj * cb + hk * cc + hl * cd
            pad[row, pl.ds(0, 16)] = acc
            acc2 = acc + pad[row, pl.ds(8, 16)]
            s = acc2[0]
            for c in range(1, 8):
                s = s + acc2[c]
            base = (row // 16) * 16
            plsc.addupdate(score_buf.at[pl.ds(base, 16)],
                           jnp.where(lane == row % 16, s, 0.0))

        pltpu.sync_copy(score_buf, out_hbm.at[pl.ds(wid * B_PER_W + j * CH, CH)])
        return 0

    lax.fori_loop(0, NCH, do_chunk, 0)


def kernel(head, relation, tail, entity_emb, relation_emb):
    return _quate_sc(head.astype(jnp.int32), relation.astype(jnp.int32),
                     tail.astype(jnp.int32), entity_emb, relation_emb)
